# baseline (speedup 1.0000x reference)
import functools

import jax
import jax.numpy as jnp
from jax import lax
from jax.experimental import pallas as pl
from jax.experimental.pallas import tpu as pltpu
from jax.experimental.pallas import tpu_sc as plsc

NUM_EMBEDDINGS = 1000000
EMBED_DIM = 64
PADDING_IDX = 0
BATCH = 4096
SEQ = 200

NC = 2
NS = 16
NW = NC * NS
BPW = BATCH // NW

NG = 8
NT = 4
GLEAD = 5
LANES = 16

_MESH = plsc.VectorSubcoreMesh(
    core_axis_name="c", subcore_axis_name="s", num_cores=NC, num_subcores=NS
)


@functools.partial(
    pl.kernel,
    out_type=jax.ShapeDtypeStruct((SEQ, EMBED_DIM, BATCH), jnp.float32),
    mesh=_MESH,
    scratch_types=[
        pltpu.VMEM((SEQ, BPW), jnp.int32),
        pltpu.VMEM((NG, BPW, EMBED_DIM), jnp.float32),
        pltpu.VMEM((NT, EMBED_DIM, BPW), jnp.float32),
        pltpu.SemaphoreType.DMA((NG,)),
        pltpu.SemaphoreType.DMA((NT,)),
    ],
    compiler_params=pltpu.CompilerParams(
        use_tc_tiling_on_sc=False, needs_layout_passes=False
    ),
)
def _gather_sc(xt_hbm, table_hbm, out_hbm, idx_v, g_v, t_v, gsem, ssem):
    wid = lax.axis_index("s") * NC + lax.axis_index("c")
    b0 = wid * BPW
    pltpu.sync_copy(xt_hbm.at[:, pl.ds(b0, BPW)], idx_v)

    def g_desc(s, g):
        return pltpu.make_async_copy(
            table_hbm.at[idx_v.at[s]], g_v.at[g], gsem.at[g]
        )

    def s_desc(s, t):
        return pltpu.make_async_copy(
            t_v.at[t], out_hbm.at[s, :, pl.ds(b0, BPW)], ssem.at[t]
        )

    iota = lax.iota(jnp.int32, LANES)
    row_ids = [iota + LANES * v for v in range(BPW // LANES)]

    def transpose(g, t):
        def body(c, carry):
            col = jnp.full((LANES,), c, jnp.int32)
            for v in range(BPW // LANES):
                vals = plsc.load_gather(g_v.at[g], [row_ids[v], col])
                t_v[t, c, pl.ds(LANES * v, LANES)] = vals
            return carry

        lax.fori_loop(0, EMBED_DIM, body, 0, unroll=2)

    for g in range(GLEAD + 1):
        g_desc(g, g).start()

    def outer(k, carry):
        s0 = k * NG
        for i in range(NG):
            s = s0 + i
            g = i
            t = i % NT
            g_desc(s, g).wait()

            @pl.when(s >= NT)
            def _():
                s_desc(s - NT, t).wait()

            transpose(g, t)
            s_desc(s, t).start()
            j = s + GLEAD + 1

            @pl.when(j < SEQ)
            def _():
                g_desc(j, (i + GLEAD + 1) % NG).start()
        return carry

    lax.fori_loop(0, SEQ // NG, outer, 0)

    for i in range(NT):
        s = SEQ - NT + i
        s_desc(s, s % NT).wait()


def _mask_body(x_ref, m_ref):
    m_ref[...] = (x_ref[...] != PADDING_IDX).astype(jnp.float32)


def _mask_tc(xt):
    return pl.pallas_call(
        _mask_body,
        out_shape=jax.ShapeDtypeStruct((SEQ, BATCH), jnp.float32),
        grid=(25,),
        in_specs=[pl.BlockSpec((8, BATCH), lambda i: (i, 0))],
        out_specs=pl.BlockSpec((8, BATCH), lambda i: (i, 0)),
    )(xt)


def kernel(x, table):
    xt = x.T
    out_t = _gather_sc(xt, table)
    mask_t = _mask_tc(xt)
    return out_t.transpose(2, 0, 1), mask_t.T


# speedup vs baseline: 1.9822x; 1.9822x over previous
import functools

import jax
import jax.numpy as jnp
from jax import lax
from jax.experimental import pallas as pl
from jax.experimental.pallas import tpu as pltpu
from jax.experimental.pallas import tpu_sc as plsc

NUM_EMBEDDINGS = 1000000
EMBED_DIM = 64
PADDING_IDX = 0
BATCH = 4096
SEQ = 200

NC = 2
NS = 16
NW = NC * NS
BPW = BATCH // NW

NG = 8
NT = 4
GLEAD = 5
LANES = 16

_MESH = plsc.VectorSubcoreMesh(
    core_axis_name="c", subcore_axis_name="s", num_cores=NC, num_subcores=NS
)


@functools.partial(
    pl.kernel,
    out_type=jax.ShapeDtypeStruct((SEQ, 8, NW, 8, BPW), jnp.float32),
    mesh=_MESH,
    scratch_types=[
        pltpu.VMEM((SEQ // 8, 8, BPW), jnp.int32),
        pltpu.VMEM((NG, BPW, EMBED_DIM), jnp.float32),
        pltpu.VMEM((NT, 8, 8, BPW), jnp.float32),
        pltpu.SemaphoreType.DMA((NG,)),
        pltpu.SemaphoreType.DMA((NT,)),
    ],
    compiler_params=pltpu.CompilerParams(
        use_tc_tiling_on_sc=False, needs_layout_passes=False
    ),
)
def _gather_sc(x5_hbm, table_hbm, out_hbm, idx_v, g_v, t_v, gsem, ssem):
    wid = lax.axis_index("s") * NC + lax.axis_index("c")
    pltpu.sync_copy(x5_hbm.at[:, wid], idx_v)

    def g_desc(s, g):
        return pltpu.make_async_copy(
            table_hbm.at[idx_v.at[s // 8, s % 8]], g_v.at[g], gsem.at[g]
        )

    def s_desc(s, t):
        return pltpu.make_async_copy(
            t_v.at[t], out_hbm.at[s, :, wid], ssem.at[t]
        )

    iota = lax.iota(jnp.int32, LANES)
    rows = [iota + LANES * v for v in range(BPW // LANES)]

    def transpose(g, t):
        def body(o, carry):
            rot = (iota + o) & (LANES - 1)
            for w in range(EMBED_DIM // LANES):
                c16 = rot + LANES * w
                tc16 = c16 >> 3
                ci16 = c16 & 7
                for v in range(BPW // LANES):
                    vals = plsc.load_gather(g_v.at[g], [rows[v], c16])
                    plsc.store_scatter(
                        t_v.at[t], [tc16, ci16, rows[v]], vals
                    )
            return carry

        lax.fori_loop(0, LANES, body, 0)

    for g in range(GLEAD + 1):
        g_desc(g, g).start()

    def outer(k, carry):
        s0 = k * NG
        for i in range(NG):
            s = s0 + i
            g = i
            t = i % NT
            g_desc(s, g).wait()

            @pl.when(s >= NT)
            def _():
                s_desc(s - NT, t).wait()

            transpose(g, t)
            s_desc(s, t).start()
            j = s + GLEAD + 1

            @pl.when(j < SEQ)
            def _():
                g_desc(j, (i + GLEAD + 1) % NG).start()
        return carry

    lax.fori_loop(0, SEQ // NG, outer, 0)

    for i in range(NT):
        s = SEQ - NT + i
        s_desc(s, s % NT).wait()


def _mask_body(x_ref, m_ref):
    m_ref[...] = (x_ref[...] != PADDING_IDX).astype(jnp.float32)


def _mask_tc(x5):
    return pl.pallas_call(
        _mask_body,
        out_shape=jax.ShapeDtypeStruct((SEQ // 8, NW, 8, BPW), jnp.float32),
        grid=(5,),
        in_specs=[pl.BlockSpec((5, NW, 8, BPW), lambda i: (i, 0, 0, 0))],
        out_specs=pl.BlockSpec((5, NW, 8, BPW), lambda i: (i, 0, 0, 0)),
    )(x5)


def kernel(x, table):
    x5 = x.T.reshape(SEQ // 8, 8, NW, BPW).transpose(0, 2, 1, 3)
    out5 = _gather_sc(x5, table)
    mask5 = _mask_tc(x5)
    out = out5.transpose(2, 4, 0, 1, 3).reshape(BATCH, SEQ, EMBED_DIM)
    mask = mask5.transpose(1, 3, 0, 2).reshape(BATCH, SEQ)
    return out, mask


# speedup vs baseline: 2.1356x; 1.0774x over previous
import functools

import jax
import jax.numpy as jnp
from jax import lax
from jax.experimental import pallas as pl
from jax.experimental.pallas import tpu as pltpu
from jax.experimental.pallas import tpu_sc as plsc

NUM_EMBEDDINGS = 1000000
EMBED_DIM = 64
PADDING_IDX = 0
BATCH = 4096
SEQ = 200

NC = 2
NS = 16
NW = NC * NS
BPW = BATCH // NW

NG = 4
NT = 4
GLEAD = 3
LANES = 16
PADDED = 128

_MESH = plsc.VectorSubcoreMesh(
    core_axis_name="c", subcore_axis_name="s", num_cores=NC, num_subcores=NS
)


@functools.partial(
    pl.kernel,
    out_type=jax.ShapeDtypeStruct((SEQ, 8, NW, 8, BPW), jnp.float32),
    mesh=_MESH,
    scratch_types=[
        pltpu.VMEM((SEQ // 8, 8, BPW), jnp.int32),
        pltpu.VMEM((NG, BPW, PADDED), jnp.float32),
        pltpu.VMEM((NT, 8, 8, BPW), jnp.float32),
        pltpu.SemaphoreType.DMA((NG,)),
        pltpu.SemaphoreType.DMA((NT,)),
    ],
    compiler_params=pltpu.CompilerParams(
        use_tc_tiling_on_sc=False,
        needs_layout_passes=False,
        disable_bounds_checks=True,
    ),
)
def _gather_sc(x5_hbm, table_hbm, out_hbm, idx_v, g_v, t_v, gsem, ssem):
    wid = lax.axis_index("s") * NC + lax.axis_index("c")
    pltpu.sync_copy(x5_hbm.at[:, wid], idx_v)

    def g_desc(s, g):
        return pltpu.make_async_copy(
            table_hbm.at[idx_v.at[s // 8, s % 8]],
            g_v.at[g],
            gsem.at[g],
        )

    def s_desc(s, t):
        return pltpu.make_async_copy(
            t_v.at[t], out_hbm.at[s, :, wid], ssem.at[t]
        )

    iota = lax.iota(jnp.int32, LANES)
    rows = [iota + LANES * v for v in range(BPW // LANES)]

    def transpose(g, t):
        def body(o, carry):
            rot = (iota + o) & (LANES - 1)
            for w in range(EMBED_DIM // LANES):
                c16 = rot + LANES * w
                tc16 = c16 >> 3
                ci16 = c16 & 7
                for v in range(BPW // LANES):
                    vals = plsc.load_gather(g_v.at[g], [rows[v], c16])
                    plsc.store_scatter(
                        t_v.at[t], [tc16, ci16, rows[v]], vals
                    )
            return carry

        lax.fori_loop(0, LANES, body, 0)

    for g in range(GLEAD + 1):
        g_desc(g, g).start()

    def outer(k, carry):
        s0 = k * NG
        for i in range(NG):
            s = s0 + i
            g = i
            t = i % NT
            g_desc(s, g).wait()

            @pl.when(s >= NT)
            def _():
                s_desc(s - NT, t).wait()

            transpose(g, t)
            s_desc(s, t).start()
            j = s + GLEAD + 1

            @pl.when(j < SEQ)
            def _():
                g_desc(j, (i + GLEAD + 1) % NG).start()
        return carry

    lax.fori_loop(0, SEQ // NG, outer, 0)

    for i in range(NT):
        s = SEQ - NT + i
        s_desc(s, s % NT).wait()


def _mask_body(x_ref, m_ref):
    m_ref[...] = (x_ref[...] != PADDING_IDX).astype(jnp.float32)


def _mask_tc(x5):
    return pl.pallas_call(
        _mask_body,
        out_shape=jax.ShapeDtypeStruct((SEQ // 8, NW, 8, BPW), jnp.float32),
        grid=(5,),
        in_specs=[pl.BlockSpec((5, NW, 8, BPW), lambda i: (i, 0, 0, 0))],
        out_specs=pl.BlockSpec((5, NW, 8, BPW), lambda i: (i, 0, 0, 0)),
    )(x5)


def kernel(x, table):
    x5 = x.T.reshape(SEQ // 8, 8, NW, BPW).transpose(0, 2, 1, 3)
    t2 = jnp.pad(table, ((0, 0), (0, PADDED - EMBED_DIM)))
    out5 = _gather_sc(x5, t2)
    mask5 = _mask_tc(x5)
    out = out5.transpose(2, 4, 0, 1, 3).reshape(BATCH, SEQ, EMBED_DIM)
    mask = mask5.transpose(1, 3, 0, 2).reshape(BATCH, SEQ)
    return out, mask


# speedup vs baseline: 2.6585x; 1.2448x over previous
import functools

import jax
import jax.numpy as jnp
from jax import lax
from jax.experimental import pallas as pl
from jax.experimental.pallas import tpu as pltpu
from jax.experimental.pallas import tpu_sc as plsc

NUM_EMBEDDINGS = 1000000
EMBED_DIM = 64
PADDING_IDX = 0
BATCH = 4096
SEQ = 200

NC = 2
NS = 16
NW = NC * NS
BPW = BATCH // NW

NG = 4
NT = 4
GLEAD = 3
LANES = 16
PADDED = 128

_MESH = plsc.VectorSubcoreMesh(
    core_axis_name="c", subcore_axis_name="s", num_cores=NC, num_subcores=NS
)


@functools.partial(
    pl.kernel,
    out_type=jax.ShapeDtypeStruct((SEQ, 8, NW, 8, BPW), jnp.float32),
    mesh=_MESH,
    scratch_types=[
        pltpu.VMEM((SEQ // 8, 8, BPW), jnp.int32),
        pltpu.VMEM((NG, BPW, PADDED), jnp.float32),
        pltpu.VMEM((NT, 8, 8, BPW), jnp.float32),
        pltpu.SemaphoreType.DMA((NG,)),
        pltpu.SemaphoreType.DMA((NT,)),
    ],
    compiler_params=pltpu.CompilerParams(
        use_tc_tiling_on_sc=False,
        needs_layout_passes=False,
        disable_bounds_checks=True,
    ),
)
def _gather_sc(x5_hbm, table_hbm, out_hbm, idx_v, g_v, t_v, gsem, ssem):
    wid = lax.axis_index("s") * NC + lax.axis_index("c")
    pltpu.sync_copy(x5_hbm.at[:, wid], idx_v)

    def g_desc(s, g):
        return pltpu.make_async_copy(
            table_hbm.at[idx_v.at[s // 8, s % 8]],
            g_v.at[g],
            gsem.at[g],
        )

    def s_desc(s, t):
        return pltpu.make_async_copy(
            t_v.at[t], out_hbm.at[s, :, wid], ssem.at[t]
        )

    iota = lax.iota(jnp.int32, LANES)
    rows = [iota + LANES * v for v in range(BPW // LANES)]

    def transpose(g, t):
        @plsc.parallel_loop(0, LANES, unroll=2)
        def _(o):
            rot = (iota + o) & (LANES - 1)
            for w in range(EMBED_DIM // LANES):
                c16 = rot + LANES * w
                tc16 = c16 >> 3
                ci16 = c16 & 7
                for v in range(BPW // LANES):
                    vals = plsc.load_gather(g_v.at[g], [rows[v], c16])
                    plsc.store_scatter(
                        t_v.at[t], [tc16, ci16, rows[v]], vals
                    )

    for g in range(GLEAD + 1):
        g_desc(g, g).start()

    def outer(k, carry):
        s0 = k * NG
        for i in range(NG):
            s = s0 + i
            g = i
            t = i % NT
            g_desc(s, g).wait()

            @pl.when(s >= NT)
            def _():
                s_desc(s - NT, t).wait()

            transpose(g, t)
            s_desc(s, t).start()
            j = s + GLEAD + 1

            @pl.when(j < SEQ)
            def _():
                g_desc(j, (i + GLEAD + 1) % NG).start()
        return carry

    lax.fori_loop(0, SEQ // NG, outer, 0)

    for i in range(NT):
        s = SEQ - NT + i
        s_desc(s, s % NT).wait()


def _mask_body(x_ref, m_ref):
    m_ref[...] = (x_ref[...] != PADDING_IDX).astype(jnp.float32)


def _mask_tc(x5):
    return pl.pallas_call(
        _mask_body,
        out_shape=jax.ShapeDtypeStruct((SEQ // 8, NW, 8, BPW), jnp.float32),
        grid=(5,),
        in_specs=[pl.BlockSpec((5, NW, 8, BPW), lambda i: (i, 0, 0, 0))],
        out_specs=pl.BlockSpec((5, NW, 8, BPW), lambda i: (i, 0, 0, 0)),
    )(x5)


def kernel(x, table):
    x5 = x.T.reshape(SEQ // 8, 8, NW, BPW).transpose(0, 2, 1, 3)
    t2 = jnp.pad(table, ((0, 0), (0, PADDED - EMBED_DIM)))
    out5 = _gather_sc(x5, t2)
    mask5 = _mask_tc(x5)
    out = out5.transpose(2, 4, 0, 1, 3).reshape(BATCH, SEQ, EMBED_DIM)
    mask = mask5.transpose(1, 3, 0, 2).reshape(BATCH, SEQ)
    return out, mask
